# baseline (speedup 1.0000x reference)
import functools

import jax
import jax.numpy as jnp
from jax.experimental import pallas as pl
from jax.experimental.pallas import tpu as pltpu
from jax.experimental.pallas import tpu_sc as plsc

_D = 64
_D2 = 128
_M = 16
_NB = 400
_CB = 400
_EPS = 1e-5


def _softplus(x):
    return jnp.maximum(x, 0.0) + jnp.log1p(jnp.exp(-jnp.abs(x)))


def _bn_scale(s_ref, q_ref, g_ref, b_ref, count):
    s = jnp.sum(s_ref[:, 0, :], axis=0, keepdims=True)
    q = jnp.sum(q_ref[:, 0, :], axis=0, keepdims=True)
    mu = s / count
    var = q / count - mu * mu
    inv = g_ref[...] * jax.lax.rsqrt(var + _EPS)
    return inv, b_ref[...] - mu * inv


def _sc_gather(table, idx, window):
    k = idx.shape[0]
    d = table.shape[1]
    idx2 = idx.reshape(1, k)
    mesh = plsc.VectorSubcoreMesh(core_axis_name="core", subcore_axis_name="subcore")

    @functools.partial(
        pl.kernel,
        out_type=jax.ShapeDtypeStruct((k, d), table.dtype),
        mesh=mesh,
        compiler_params=pltpu.CompilerParams(use_tc_tiling_on_sc=False),
    )
    def gather_kernel(tbl_hbm, idx_hbm, out_hbm):
        def body(i_vmem, o_vmem):
            pltpu.sync_copy(tbl_hbm.at[i_vmem.at[0]], o_vmem)

        pltpu.emit_pipeline(
            body,
            grid=(k // window,),
            in_specs=[pl.BlockSpec((1, window), lambda i: (0, i))],
            out_specs=[pl.BlockSpec((window, d), lambda i: (i, 0))],
            core_axis_name=("core", "subcore"),
            dimension_semantics=(pltpu.PARALLEL,),
        )(idx_hbm, out_hbm)

    return gather_kernel(table, idx2)


def _embed(a, w, b):
    n, fin = a.shape
    d = w.shape[1]

    def body(a_ref, w_ref, b_ref, x_ref):
        x_ref[...] = (
            jnp.dot(a_ref[...], w_ref[...], preferred_element_type=jnp.float32)
            + b_ref[...]
        )

    return pl.pallas_call(
        body,
        grid=(n // _NB,),
        in_specs=[
            pl.BlockSpec((_NB, fin), lambda i: (i, 0)),
            pl.BlockSpec((fin, d), lambda i: (0, 0)),
            pl.BlockSpec((1, d), lambda i: (0, 0)),
        ],
        out_specs=pl.BlockSpec((_NB, d), lambda i: (i, 0)),
        out_shape=jax.ShapeDtypeStruct((n, d), jnp.float32),
        compiler_params=pltpu.CompilerParams(dimension_semantics=("parallel",)),
    )(a, w, b)


def _conv_stats(g3, ef2, x, ws, wnb, wbig):
    n = x.shape[0]
    grid = n // _NB

    def body(g_ref, ef_ref, x_ref, ws_ref, wn_ref, wb_ref, s_ref, q_ref):
        u = jnp.dot(x_ref[...], ws_ref[...], preferred_element_type=jnp.float32)
        efc = jnp.dot(ef_ref[...], wb_ref[...], preferred_element_type=jnp.float32)
        s = jnp.zeros((1, _D2), jnp.float32)
        q = jnp.zeros((1, _D2), jnp.float32)
        for m in range(_M):
            nb = jnp.dot(
                g_ref[m].astype(jnp.bfloat16), wn_ref[...],
                preferred_element_type=jnp.float32,
            )
            pre = nb + u + efc[:, m * _D2:(m + 1) * _D2]
            s = s + jnp.sum(pre, axis=0, keepdims=True)
            q = q + jnp.sum(pre * pre, axis=0, keepdims=True)
        s_ref[0] = s
        q_ref[0] = q

    return pl.pallas_call(
        body,
        grid=(grid,),
        in_specs=[
            pl.BlockSpec((_M, _NB, _D), lambda i: (0, i, 0)),
            pl.BlockSpec((_NB, _M * 16), lambda i: (i, 0)),
            pl.BlockSpec((_NB, _D), lambda i: (i, 0)),
            pl.BlockSpec((_D, _D2), lambda i: (0, 0)),
            pl.BlockSpec((_D, _D2), lambda i: (0, 0)),
            pl.BlockSpec((_M * 16, _M * _D2), lambda i: (0, 0)),
        ],
        out_specs=[
            pl.BlockSpec((1, 1, _D2), lambda i: (i, 0, 0)),
            pl.BlockSpec((1, 1, _D2), lambda i: (i, 0, 0)),
        ],
        out_shape=[
            jax.ShapeDtypeStruct((grid, 1, _D2), jnp.float32),
            jax.ShapeDtypeStruct((grid, 1, _D2), jnp.float32),
        ],
        compiler_params=pltpu.CompilerParams(dimension_semantics=("parallel",)),
    )(g3, ef2, x, ws, wnb, wbig)


def _conv_apply(g3, ef2, x, ws, wnb, wbig, s_p, q_p, g1, b1, cnt):
    n = x.shape[0]
    grid = n // _NB

    def body(g_ref, ef_ref, x_ref, ws_ref, wn_ref, wb_ref, sp_ref, qp_ref,
             g1_ref, b1_ref, sum_ref, s_ref, q_ref):
        s1, t1 = _bn_scale(sp_ref, qp_ref, g1_ref, b1_ref, cnt)
        u = jnp.dot(x_ref[...], ws_ref[...], preferred_element_type=jnp.float32)
        efc = jnp.dot(ef_ref[...], wb_ref[...], preferred_element_type=jnp.float32)
        acc = jnp.zeros((_NB, _D), jnp.float32)
        for m in range(_M):
            nb = jnp.dot(
                g_ref[m].astype(jnp.bfloat16), wn_ref[...],
                preferred_element_type=jnp.float32,
            )
            pre = nb + u + efc[:, m * _D2:(m + 1) * _D2]
            z = pre * s1 + t1
            f = z[:, :_D]
            c = z[:, _D:]
            acc = acc + jax.nn.sigmoid(f) * _softplus(c)
        sum_ref[...] = acc
        s_ref[0] = jnp.sum(acc, axis=0, keepdims=True)
        q_ref[0] = jnp.sum(acc * acc, axis=0, keepdims=True)

    part = pl.BlockSpec((grid, 1, _D2), lambda i: (0, 0, 0))
    return pl.pallas_call(
        body,
        grid=(grid,),
        in_specs=[
            pl.BlockSpec((_M, _NB, _D), lambda i: (0, i, 0)),
            pl.BlockSpec((_NB, _M * 16), lambda i: (i, 0)),
            pl.BlockSpec((_NB, _D), lambda i: (i, 0)),
            pl.BlockSpec((_D, _D2), lambda i: (0, 0)),
            pl.BlockSpec((_D, _D2), lambda i: (0, 0)),
            pl.BlockSpec((_M * 16, _M * _D2), lambda i: (0, 0)),
            part, part,
            pl.BlockSpec((1, _D2), lambda i: (0, 0)),
            pl.BlockSpec((1, _D2), lambda i: (0, 0)),
        ],
        out_specs=[
            pl.BlockSpec((_NB, _D), lambda i: (i, 0)),
            pl.BlockSpec((1, 1, _D), lambda i: (i, 0, 0)),
            pl.BlockSpec((1, 1, _D), lambda i: (i, 0, 0)),
        ],
        out_shape=[
            jax.ShapeDtypeStruct((n, _D), jnp.float32),
            jax.ShapeDtypeStruct((grid, 1, _D), jnp.float32),
            jax.ShapeDtypeStruct((grid, 1, _D), jnp.float32),
        ],
        compiler_params=pltpu.CompilerParams(dimension_semantics=("parallel",)),
    )(g3, ef2, x, ws, wnb, wbig, s_p, q_p, g1, b1)


def _residual(x, summed, s2_p, q2_p, g2, b2, count):
    n = x.shape[0]
    grid = n // _NB

    def body(x_ref, sm_ref, sp_ref, qp_ref, g2_ref, b2_ref, o_ref):
        s2, t2 = _bn_scale(sp_ref, qp_ref, g2_ref, b2_ref, count)
        o_ref[...] = _softplus(x_ref[...] + sm_ref[...] * s2 + t2)

    part = pl.BlockSpec((grid, 1, _D), lambda i: (0, 0, 0))
    return pl.pallas_call(
        body,
        grid=(grid,),
        in_specs=[
            pl.BlockSpec((_NB, _D), lambda i: (i, 0)),
            pl.BlockSpec((_NB, _D), lambda i: (i, 0)),
            part, part,
            pl.BlockSpec((1, _D), lambda i: (0, 0)),
            pl.BlockSpec((1, _D), lambda i: (0, 0)),
        ],
        out_specs=pl.BlockSpec((_NB, _D), lambda i: (i, 0)),
        out_shape=jax.ShapeDtypeStruct((n, _D), jnp.float32),
        compiler_params=pltpu.CompilerParams(dimension_semantics=("parallel",)),
    )(x, summed, s2_p, q2_p, g2, b2)


def _head(gc3, wfc, bfc, wh1, bh1, wh2, bh2):
    a, c, d = gc3.shape
    h = wfc.shape[1]

    def body(g_ref, wfc_ref, bfc_ref, wh1_ref, bh1_ref, wh2_ref, bh2_ref, o_ref):
        acc = jnp.zeros((_CB, d), jnp.float32)
        for m in range(a):
            acc = acc + g_ref[m]
        crys = _softplus(acc / jnp.float32(a))
        crys = jnp.dot(crys, wfc_ref[...], preferred_element_type=jnp.float32) + bfc_ref[...]
        crys = _softplus(crys)
        hh = _softplus(
            jnp.dot(crys, wh1_ref[...], preferred_element_type=jnp.float32) + bh1_ref[...]
        )
        o_ref[...] = (
            jnp.dot(hh, wh2_ref[...], preferred_element_type=jnp.float32) + bh2_ref[...]
        )

    return pl.pallas_call(
        body,
        grid=(c // _CB,),
        in_specs=[
            pl.BlockSpec((a, _CB, d), lambda i: (0, i, 0)),
            pl.BlockSpec((d, h), lambda i: (0, 0)),
            pl.BlockSpec((1, h), lambda i: (0, 0)),
            pl.BlockSpec((h, h), lambda i: (0, 0)),
            pl.BlockSpec((1, h), lambda i: (0, 0)),
            pl.BlockSpec((h, h), lambda i: (0, 0)),
            pl.BlockSpec((1, h), lambda i: (0, 0)),
        ],
        out_specs=pl.BlockSpec((_CB, h), lambda i: (i, 0)),
        out_shape=jax.ShapeDtypeStruct((c, h), jnp.float32),
        compiler_params=pltpu.CompilerParams(dimension_semantics=("parallel",)),
    )(gc3, wfc, bfc, wh1, bh1, wh2, bh2)


def kernel(atomic_features, num_features, feature_index, crystal_index, params):
    p = params
    n, m = feature_index.shape
    cnt = float(n * m)
    convs = p["convs"]

    fi_t = feature_index.astype(jnp.int32).T.reshape(-1)
    ci_t = crystal_index.astype(jnp.int32).T.reshape(-1)
    ef2 = num_features.reshape(n, m * 16).astype(jnp.bfloat16)
    eye_m = jnp.eye(m, dtype=jnp.float32)

    x = _embed(atomic_features, p["W_emb"], p["b_emb"].reshape(1, -1))

    for cp in convs:
        w = cp["W"]
        ws, we = w[:_D], w[2 * _D:]
        wnb = w[_D:2 * _D].astype(jnp.bfloat16)
        wbig = jnp.kron(eye_m, we).astype(jnp.bfloat16)

        g3 = _sc_gather(x, fi_t, 128).reshape(m, n, _D)

        s_p, q_p = _conv_stats(g3, ef2, x, ws, wnb, wbig)
        summed, s2_p, q2_p = _conv_apply(
            g3, ef2, x, ws, wnb, wbig, s_p, q_p,
            cp["g1"].reshape(1, -1), cp["b1"].reshape(1, -1), cnt,
        )
        x = _residual(
            x, summed, s2_p, q2_p,
            cp["g2"].reshape(1, -1), cp["b2"].reshape(1, -1), float(n),
        )

    c, a = crystal_index.shape
    gc3 = _sc_gather(x, ci_t, 80).reshape(a, c, _D)
    return _head(
        gc3,
        p["W_fc"], p["b_fc"].reshape(1, -1),
        p["W_h1"], p["b_h1"].reshape(1, -1),
        p["W_h2"], p["b_h2"].reshape(1, -1),
    )


# speedup vs baseline: 1.2242x; 1.2242x over previous
import functools

import jax
import jax.numpy as jnp
from jax.experimental import pallas as pl
from jax.experimental.pallas import tpu as pltpu
from jax.experimental.pallas import tpu_sc as plsc

_D = 64
_D2 = 128
_M = 16
_NB = 400
_CB = 400
_EPS = 1e-5


def _softplus(x):
    return jnp.maximum(x, 0.0) + jnp.log1p(jnp.exp(-jnp.abs(x)))


def _bn_scale(s_ref, q_ref, g_ref, b_ref, count):
    s = jnp.sum(s_ref[:, 0, :], axis=0, keepdims=True)
    q = jnp.sum(q_ref[:, 0, :], axis=0, keepdims=True)
    mu = s / count
    var = q / count - mu * mu
    inv = g_ref[...] * jax.lax.rsqrt(var + _EPS)
    return inv, b_ref[...] - mu * inv


def _sc_gather(table, idx, window, tc_tiling):
    k = idx.shape[0]
    d = table.shape[1]
    idx2 = idx.reshape(1, k)
    mesh = plsc.VectorSubcoreMesh(core_axis_name="core", subcore_axis_name="subcore")

    @functools.partial(
        pl.kernel,
        out_type=jax.ShapeDtypeStruct((k, d), table.dtype),
        mesh=mesh,
        compiler_params=pltpu.CompilerParams(use_tc_tiling_on_sc=tc_tiling),
    )
    def gather_kernel(tbl_hbm, idx_hbm, out_hbm):
        def body(i_vmem, o_vmem):
            pltpu.sync_copy(tbl_hbm.at[i_vmem.at[0]], o_vmem)

        pltpu.emit_pipeline(
            body,
            grid=(k // window,),
            in_specs=[pl.BlockSpec((1, window), lambda i: (0, i))],
            out_specs=[pl.BlockSpec((window, d), lambda i: (i, 0))],
            core_axis_name=("core", "subcore"),
            dimension_semantics=(pltpu.PARALLEL,),
        )(idx_hbm, out_hbm)

    return gather_kernel(table, idx2)


def _embed(a, w, b, wn0):
    n, fin = a.shape
    d = w.shape[1]

    def body(a_ref, w_ref, b_ref, wn_ref, x_ref, v_ref):
        x = jnp.dot(a_ref[...], w_ref[...], preferred_element_type=jnp.float32)
        x = x + b_ref[...]
        x_ref[...] = x
        v_ref[...] = jnp.dot(x, wn_ref[...], preferred_element_type=jnp.float32)

    return pl.pallas_call(
        body,
        grid=(n // _NB,),
        in_specs=[
            pl.BlockSpec((_NB, fin), lambda i: (i, 0)),
            pl.BlockSpec((fin, d), lambda i: (0, 0)),
            pl.BlockSpec((1, d), lambda i: (0, 0)),
            pl.BlockSpec((d, _D2), lambda i: (0, 0)),
        ],
        out_specs=[
            pl.BlockSpec((_NB, d), lambda i: (i, 0)),
            pl.BlockSpec((_NB, _D2), lambda i: (i, 0)),
        ],
        out_shape=[
            jax.ShapeDtypeStruct((n, d), jnp.float32),
            jax.ShapeDtypeStruct((n, _D2), jnp.float32),
        ],
        compiler_params=pltpu.CompilerParams(dimension_semantics=("parallel",)),
    )(a, w, b, wn0)


def _conv_stats(gv3, ef2, x, ws, wbig):
    n = x.shape[0]
    grid = n // _NB

    def body(g_ref, ef_ref, x_ref, ws_ref, wb_ref, s_ref, q_ref):
        u = jnp.dot(x_ref[...], ws_ref[...], preferred_element_type=jnp.float32)
        efc = jnp.dot(ef_ref[...], wb_ref[...], preferred_element_type=jnp.float32)
        s = jnp.zeros((1, _D2), jnp.float32)
        q = jnp.zeros((1, _D2), jnp.float32)
        for m in range(_M):
            pre = g_ref[m] + u + efc[:, m * _D2:(m + 1) * _D2]
            s = s + jnp.sum(pre, axis=0, keepdims=True)
            q = q + jnp.sum(pre * pre, axis=0, keepdims=True)
        s_ref[0] = s
        q_ref[0] = q

    return pl.pallas_call(
        body,
        grid=(grid,),
        in_specs=[
            pl.BlockSpec((_M, _NB, _D2), lambda i: (0, i, 0)),
            pl.BlockSpec((_NB, _M * 16), lambda i: (i, 0)),
            pl.BlockSpec((_NB, _D), lambda i: (i, 0)),
            pl.BlockSpec((_D, _D2), lambda i: (0, 0)),
            pl.BlockSpec((_M * 16, _M * _D2), lambda i: (0, 0)),
        ],
        out_specs=[
            pl.BlockSpec((1, 1, _D2), lambda i: (i, 0, 0)),
            pl.BlockSpec((1, 1, _D2), lambda i: (i, 0, 0)),
        ],
        out_shape=[
            jax.ShapeDtypeStruct((grid, 1, _D2), jnp.float32),
            jax.ShapeDtypeStruct((grid, 1, _D2), jnp.float32),
        ],
        compiler_params=pltpu.CompilerParams(dimension_semantics=("parallel",)),
    )(gv3, ef2, x, ws, wbig)


def _conv_apply(gv3, ef2, x, ws, wbig, s_p, q_p, g1, b1, cnt):
    n = x.shape[0]
    grid = n // _NB

    def body(g_ref, ef_ref, x_ref, ws_ref, wb_ref, sp_ref, qp_ref,
             g1_ref, b1_ref, sum_ref, s_ref, q_ref):
        s1, t1 = _bn_scale(sp_ref, qp_ref, g1_ref, b1_ref, cnt)
        u = jnp.dot(x_ref[...], ws_ref[...], preferred_element_type=jnp.float32)
        efc = jnp.dot(ef_ref[...], wb_ref[...], preferred_element_type=jnp.float32)
        acc = jnp.zeros((_NB, _D), jnp.float32)
        for m in range(_M):
            pre = g_ref[m] + u + efc[:, m * _D2:(m + 1) * _D2]
            z = pre * s1 + t1
            f = z[:, :_D]
            c = z[:, _D:]
            acc = acc + jax.nn.sigmoid(f) * _softplus(c)
        sum_ref[...] = acc
        s_ref[0] = jnp.sum(acc, axis=0, keepdims=True)
        q_ref[0] = jnp.sum(acc * acc, axis=0, keepdims=True)

    part = pl.BlockSpec((grid, 1, _D2), lambda i: (0, 0, 0))
    return pl.pallas_call(
        body,
        grid=(grid,),
        in_specs=[
            pl.BlockSpec((_M, _NB, _D2), lambda i: (0, i, 0)),
            pl.BlockSpec((_NB, _M * 16), lambda i: (i, 0)),
            pl.BlockSpec((_NB, _D), lambda i: (i, 0)),
            pl.BlockSpec((_D, _D2), lambda i: (0, 0)),
            pl.BlockSpec((_M * 16, _M * _D2), lambda i: (0, 0)),
            part, part,
            pl.BlockSpec((1, _D2), lambda i: (0, 0)),
            pl.BlockSpec((1, _D2), lambda i: (0, 0)),
        ],
        out_specs=[
            pl.BlockSpec((_NB, _D), lambda i: (i, 0)),
            pl.BlockSpec((1, 1, _D), lambda i: (i, 0, 0)),
            pl.BlockSpec((1, 1, _D), lambda i: (i, 0, 0)),
        ],
        out_shape=[
            jax.ShapeDtypeStruct((n, _D), jnp.float32),
            jax.ShapeDtypeStruct((grid, 1, _D), jnp.float32),
            jax.ShapeDtypeStruct((grid, 1, _D), jnp.float32),
        ],
        compiler_params=pltpu.CompilerParams(dimension_semantics=("parallel",)),
    )(gv3, ef2, x, ws, wbig, s_p, q_p, g1, b1)


def _residual(x, summed, s2_p, q2_p, g2, b2, count, wn_next):
    n = x.shape[0]
    grid = n // _NB
    with_v = wn_next is not None

    def body(x_ref, sm_ref, sp_ref, qp_ref, g2_ref, b2_ref, *rest):
        s2, t2 = _bn_scale(sp_ref, qp_ref, g2_ref, b2_ref, count)
        xn = _softplus(x_ref[...] + sm_ref[...] * s2 + t2)
        if with_v:
            wn_ref, o_ref, v_ref = rest
            o_ref[...] = xn
            v_ref[...] = jnp.dot(xn, wn_ref[...], preferred_element_type=jnp.float32)
        else:
            (o_ref,) = rest
            o_ref[...] = xn

    part = pl.BlockSpec((grid, 1, _D), lambda i: (0, 0, 0))
    in_specs = [
        pl.BlockSpec((_NB, _D), lambda i: (i, 0)),
        pl.BlockSpec((_NB, _D), lambda i: (i, 0)),
        part, part,
        pl.BlockSpec((1, _D), lambda i: (0, 0)),
        pl.BlockSpec((1, _D), lambda i: (0, 0)),
    ]
    out_specs = [pl.BlockSpec((_NB, _D), lambda i: (i, 0))]
    out_shape = [jax.ShapeDtypeStruct((n, _D), jnp.float32)]
    args = [x, summed, s2_p, q2_p, g2, b2]
    if with_v:
        in_specs.append(pl.BlockSpec((_D, _D2), lambda i: (0, 0)))
        out_specs.append(pl.BlockSpec((_NB, _D2), lambda i: (i, 0)))
        out_shape.append(jax.ShapeDtypeStruct((n, _D2), jnp.float32))
        args.append(wn_next)

    out = pl.pallas_call(
        body,
        grid=(grid,),
        in_specs=in_specs,
        out_specs=out_specs,
        out_shape=out_shape,
        compiler_params=pltpu.CompilerParams(dimension_semantics=("parallel",)),
    )(*args)
    return out if with_v else out[0]


def _head(gc3, wfc, bfc, wh1, bh1, wh2, bh2):
    a, c, d = gc3.shape
    h = wfc.shape[1]

    def body(g_ref, wfc_ref, bfc_ref, wh1_ref, bh1_ref, wh2_ref, bh2_ref, o_ref):
        acc = jnp.zeros((_CB, d), jnp.float32)
        for m in range(a):
            acc = acc + g_ref[m]
        crys = _softplus(acc / jnp.float32(a))
        crys = jnp.dot(crys, wfc_ref[...], preferred_element_type=jnp.float32) + bfc_ref[...]
        crys = _softplus(crys)
        hh = _softplus(
            jnp.dot(crys, wh1_ref[...], preferred_element_type=jnp.float32) + bh1_ref[...]
        )
        o_ref[...] = (
            jnp.dot(hh, wh2_ref[...], preferred_element_type=jnp.float32) + bh2_ref[...]
        )

    return pl.pallas_call(
        body,
        grid=(c // _CB,),
        in_specs=[
            pl.BlockSpec((a, _CB, d), lambda i: (0, i, 0)),
            pl.BlockSpec((d, h), lambda i: (0, 0)),
            pl.BlockSpec((1, h), lambda i: (0, 0)),
            pl.BlockSpec((h, h), lambda i: (0, 0)),
            pl.BlockSpec((1, h), lambda i: (0, 0)),
            pl.BlockSpec((h, h), lambda i: (0, 0)),
            pl.BlockSpec((1, h), lambda i: (0, 0)),
        ],
        out_specs=pl.BlockSpec((_CB, h), lambda i: (i, 0)),
        out_shape=jax.ShapeDtypeStruct((c, h), jnp.float32),
        compiler_params=pltpu.CompilerParams(dimension_semantics=("parallel",)),
    )(gc3, wfc, bfc, wh1, bh1, wh2, bh2)


def kernel(atomic_features, num_features, feature_index, crystal_index, params):
    p = params
    n, m = feature_index.shape
    cnt = float(n * m)
    convs = p["convs"]

    fi_t = feature_index.astype(jnp.int32).T.reshape(-1)
    ci_t = crystal_index.astype(jnp.int32).T.reshape(-1)
    ef2 = num_features.reshape(n, m * 16).astype(jnp.bfloat16)
    eye_m = jnp.eye(m, dtype=jnp.float32)

    x, vb = _embed(
        atomic_features, p["W_emb"], p["b_emb"].reshape(1, -1),
        convs[0]["W"][_D:2 * _D],
    )

    for li, cp in enumerate(convs):
        w = cp["W"]
        ws, we = w[:_D], w[2 * _D:]
        wbig = jnp.kron(eye_m, we).astype(jnp.bfloat16)

        gv3 = _sc_gather(vb, fi_t, 128, True).reshape(m, n, _D2)

        s_p, q_p = _conv_stats(gv3, ef2, x, ws, wbig)
        summed, s2_p, q2_p = _conv_apply(
            gv3, ef2, x, ws, wbig, s_p, q_p,
            cp["g1"].reshape(1, -1), cp["b1"].reshape(1, -1), cnt,
        )
        wn_next = convs[li + 1]["W"][_D:2 * _D] if li + 1 < len(convs) else None
        res = _residual(
            x, summed, s2_p, q2_p,
            cp["g2"].reshape(1, -1), cp["b2"].reshape(1, -1), float(n),
            wn_next,
        )
        if wn_next is not None:
            x, vb = res
        else:
            x = res

    c, a = crystal_index.shape
    gc3 = _sc_gather(x, ci_t, 80, False).reshape(a, c, _D)
    return _head(
        gc3,
        p["W_fc"], p["b_fc"].reshape(1, -1),
        p["W_h1"], p["b_h1"].reshape(1, -1),
        p["W_h2"], p["b_h2"].reshape(1, -1),
    )


# speedup vs baseline: 1.3824x; 1.1292x over previous
import functools

import jax
import jax.numpy as jnp
from jax.experimental import pallas as pl
from jax.experimental.pallas import tpu as pltpu
from jax.experimental.pallas import tpu_sc as plsc

_D = 64
_D2 = 128
_M = 16
_NB = 1000
_CB = 400
_EPS = 1e-5


def _softplus(x):
    return jnp.maximum(x, 0.0) + jnp.log1p(jnp.exp(-jnp.abs(x)))


def _bn_scale(s_ref, q_ref, g_ref, b_ref, count):
    s = jnp.sum(s_ref[:, 0, :], axis=0, keepdims=True)
    q = jnp.sum(q_ref[:, 0, :], axis=0, keepdims=True)
    mu = s / count
    var = q / count - mu * mu
    inv = g_ref[...] * jax.lax.rsqrt(var + _EPS)
    return inv, b_ref[...] - mu * inv


def _sc_gather(table, idx, window, tc_tiling):
    k = idx.shape[0]
    d = table.shape[1]
    idx2 = idx.reshape(1, k)
    mesh = plsc.VectorSubcoreMesh(core_axis_name="core", subcore_axis_name="subcore")

    @functools.partial(
        pl.kernel,
        out_type=jax.ShapeDtypeStruct((k, d), table.dtype),
        mesh=mesh,
        compiler_params=pltpu.CompilerParams(use_tc_tiling_on_sc=tc_tiling),
    )
    def gather_kernel(tbl_hbm, idx_hbm, out_hbm):
        def body(i_vmem, o_vmem):
            pltpu.sync_copy(tbl_hbm.at[i_vmem.at[0]], o_vmem)

        pltpu.emit_pipeline(
            body,
            grid=(k // window,),
            in_specs=[pl.BlockSpec((1, window), lambda i: (0, i))],
            out_specs=[pl.BlockSpec((window, d), lambda i: (i, 0))],
            core_axis_name=("core", "subcore"),
            dimension_semantics=(pltpu.PARALLEL,),
        )(idx_hbm, out_hbm)

    return gather_kernel(table, idx2)


def _embed(a, w, b, wn0):
    n, fin = a.shape
    d = w.shape[1]

    def body(a_ref, w_ref, b_ref, wn_ref, x_ref, v_ref):
        x = jnp.dot(a_ref[...], w_ref[...], preferred_element_type=jnp.float32)
        x = x + b_ref[...]
        x_ref[...] = x
        v_ref[...] = jnp.dot(x, wn_ref[...], preferred_element_type=jnp.float32)

    return pl.pallas_call(
        body,
        grid=(n // _NB,),
        in_specs=[
            pl.BlockSpec((_NB, fin), lambda i: (i, 0)),
            pl.BlockSpec((fin, d), lambda i: (0, 0)),
            pl.BlockSpec((1, d), lambda i: (0, 0)),
            pl.BlockSpec((d, _D2), lambda i: (0, 0)),
        ],
        out_specs=[
            pl.BlockSpec((_NB, d), lambda i: (i, 0)),
            pl.BlockSpec((_NB, _D2), lambda i: (i, 0)),
        ],
        out_shape=[
            jax.ShapeDtypeStruct((n, d), jnp.float32),
            jax.ShapeDtypeStruct((n, _D2), jnp.float32),
        ],
        compiler_params=pltpu.CompilerParams(dimension_semantics=("parallel",)),
    )(a, w, b, wn0)


def _conv_stats(gv3, ef2, x, ws, wbig):
    n = x.shape[0]
    grid = n // _NB

    def body(g_ref, ef_ref, x_ref, ws_ref, wb_ref, s_ref, q_ref):
        u = jnp.dot(x_ref[...], ws_ref[...], preferred_element_type=jnp.float32)
        efc = jnp.dot(ef_ref[...], wb_ref[...], preferred_element_type=jnp.float32)
        s = jnp.zeros((1, _D2), jnp.float32)
        q = jnp.zeros((1, _D2), jnp.float32)
        for m in range(_M):
            pre = g_ref[m] + u + efc[:, m * _D2:(m + 1) * _D2]
            s = s + jnp.sum(pre, axis=0, keepdims=True)
            q = q + jnp.sum(pre * pre, axis=0, keepdims=True)
        s_ref[0] = s
        q_ref[0] = q

    return pl.pallas_call(
        body,
        grid=(grid,),
        in_specs=[
            pl.BlockSpec((_M, _NB, _D2), lambda i: (0, i, 0)),
            pl.BlockSpec((_NB, _M * 16), lambda i: (i, 0)),
            pl.BlockSpec((_NB, _D), lambda i: (i, 0)),
            pl.BlockSpec((_D, _D2), lambda i: (0, 0)),
            pl.BlockSpec((_M * 16, _M * _D2), lambda i: (0, 0)),
        ],
        out_specs=[
            pl.BlockSpec((1, 1, _D2), lambda i: (i, 0, 0)),
            pl.BlockSpec((1, 1, _D2), lambda i: (i, 0, 0)),
        ],
        out_shape=[
            jax.ShapeDtypeStruct((grid, 1, _D2), jnp.float32),
            jax.ShapeDtypeStruct((grid, 1, _D2), jnp.float32),
        ],
        compiler_params=pltpu.CompilerParams(dimension_semantics=("parallel",)),
    )(gv3, ef2, x, ws, wbig)


def _conv_apply(gv3, ef2, x, ws, wbig, s_p, q_p, g1, b1, cnt):
    n = x.shape[0]
    grid = n // _NB

    def body(g_ref, ef_ref, x_ref, ws_ref, wb_ref, sp_ref, qp_ref,
             g1_ref, b1_ref, sum_ref, s_ref, q_ref):
        s1, t1 = _bn_scale(sp_ref, qp_ref, g1_ref, b1_ref, cnt)
        u = jnp.dot(x_ref[...], ws_ref[...], preferred_element_type=jnp.float32)
        efc = jnp.dot(ef_ref[...], wb_ref[...], preferred_element_type=jnp.float32)
        acc = jnp.zeros((_NB, _D), jnp.float32)
        for m in range(_M):
            pre = g_ref[m] + u + efc[:, m * _D2:(m + 1) * _D2]
            z = pre * s1 + t1
            f = z[:, :_D]
            c = z[:, _D:]
            acc = acc + jax.nn.sigmoid(f) * _softplus(c)
        sum_ref[...] = acc
        s_ref[0] = jnp.sum(acc, axis=0, keepdims=True)
        q_ref[0] = jnp.sum(acc * acc, axis=0, keepdims=True)

    part = pl.BlockSpec((grid, 1, _D2), lambda i: (0, 0, 0))
    return pl.pallas_call(
        body,
        grid=(grid,),
        in_specs=[
            pl.BlockSpec((_M, _NB, _D2), lambda i: (0, i, 0)),
            pl.BlockSpec((_NB, _M * 16), lambda i: (i, 0)),
            pl.BlockSpec((_NB, _D), lambda i: (i, 0)),
            pl.BlockSpec((_D, _D2), lambda i: (0, 0)),
            pl.BlockSpec((_M * 16, _M * _D2), lambda i: (0, 0)),
            part, part,
            pl.BlockSpec((1, _D2), lambda i: (0, 0)),
            pl.BlockSpec((1, _D2), lambda i: (0, 0)),
        ],
        out_specs=[
            pl.BlockSpec((_NB, _D), lambda i: (i, 0)),
            pl.BlockSpec((1, 1, _D), lambda i: (i, 0, 0)),
            pl.BlockSpec((1, 1, _D), lambda i: (i, 0, 0)),
        ],
        out_shape=[
            jax.ShapeDtypeStruct((n, _D), jnp.float32),
            jax.ShapeDtypeStruct((grid, 1, _D), jnp.float32),
            jax.ShapeDtypeStruct((grid, 1, _D), jnp.float32),
        ],
        compiler_params=pltpu.CompilerParams(dimension_semantics=("parallel",)),
    )(gv3, ef2, x, ws, wbig, s_p, q_p, g1, b1)


def _residual(x, summed, s2_p, q2_p, g2, b2, count, wn_next):
    n = x.shape[0]
    grid = n // _NB
    with_v = wn_next is not None

    def body(x_ref, sm_ref, sp_ref, qp_ref, g2_ref, b2_ref, *rest):
        s2, t2 = _bn_scale(sp_ref, qp_ref, g2_ref, b2_ref, count)
        xn = _softplus(x_ref[...] + sm_ref[...] * s2 + t2)
        if with_v:
            wn_ref, o_ref, v_ref = rest
            o_ref[...] = xn
            v_ref[...] = jnp.dot(xn, wn_ref[...], preferred_element_type=jnp.float32)
        else:
            (o_ref,) = rest
            o_ref[...] = xn

    part = pl.BlockSpec((grid, 1, _D), lambda i: (0, 0, 0))
    in_specs = [
        pl.BlockSpec((_NB, _D), lambda i: (i, 0)),
        pl.BlockSpec((_NB, _D), lambda i: (i, 0)),
        part, part,
        pl.BlockSpec((1, _D), lambda i: (0, 0)),
        pl.BlockSpec((1, _D), lambda i: (0, 0)),
    ]
    out_specs = [pl.BlockSpec((_NB, _D), lambda i: (i, 0))]
    out_shape = [jax.ShapeDtypeStruct((n, _D), jnp.float32)]
    args = [x, summed, s2_p, q2_p, g2, b2]
    if with_v:
        in_specs.append(pl.BlockSpec((_D, _D2), lambda i: (0, 0)))
        out_specs.append(pl.BlockSpec((_NB, _D2), lambda i: (i, 0)))
        out_shape.append(jax.ShapeDtypeStruct((n, _D2), jnp.float32))
        args.append(wn_next)

    out = pl.pallas_call(
        body,
        grid=(grid,),
        in_specs=in_specs,
        out_specs=out_specs,
        out_shape=out_shape,
        compiler_params=pltpu.CompilerParams(dimension_semantics=("parallel",)),
    )(*args)
    return out if with_v else out[0]


def _head(gc3, wfc, bfc, wh1, bh1, wh2, bh2):
    a, c, d = gc3.shape
    h = wfc.shape[1]

    def body(g_ref, wfc_ref, bfc_ref, wh1_ref, bh1_ref, wh2_ref, bh2_ref, o_ref):
        acc = jnp.zeros((_CB, d), jnp.float32)
        for m in range(a):
            acc = acc + g_ref[m]
        crys = _softplus(acc / jnp.float32(a))
        crys = jnp.dot(crys, wfc_ref[...], preferred_element_type=jnp.float32) + bfc_ref[...]
        crys = _softplus(crys)
        hh = _softplus(
            jnp.dot(crys, wh1_ref[...], preferred_element_type=jnp.float32) + bh1_ref[...]
        )
        o_ref[...] = (
            jnp.dot(hh, wh2_ref[...], preferred_element_type=jnp.float32) + bh2_ref[...]
        )

    return pl.pallas_call(
        body,
        grid=(c // _CB,),
        in_specs=[
            pl.BlockSpec((a, _CB, d), lambda i: (0, i, 0)),
            pl.BlockSpec((d, h), lambda i: (0, 0)),
            pl.BlockSpec((1, h), lambda i: (0, 0)),
            pl.BlockSpec((h, h), lambda i: (0, 0)),
            pl.BlockSpec((1, h), lambda i: (0, 0)),
            pl.BlockSpec((h, h), lambda i: (0, 0)),
            pl.BlockSpec((1, h), lambda i: (0, 0)),
        ],
        out_specs=pl.BlockSpec((_CB, h), lambda i: (i, 0)),
        out_shape=jax.ShapeDtypeStruct((c, h), jnp.float32),
        compiler_params=pltpu.CompilerParams(dimension_semantics=("parallel",)),
    )(gc3, wfc, bfc, wh1, bh1, wh2, bh2)


def kernel(atomic_features, num_features, feature_index, crystal_index, params):
    p = params
    n, m = feature_index.shape
    cnt = float(n * m)
    convs = p["convs"]

    fi_t = feature_index.astype(jnp.int32).T.reshape(-1)
    ci_t = crystal_index.astype(jnp.int32).T.reshape(-1)
    ef2 = num_features.reshape(n, m * 16).astype(jnp.bfloat16)
    eye_m = jnp.eye(m, dtype=jnp.float32)

    x, vb = _embed(
        atomic_features, p["W_emb"], p["b_emb"].reshape(1, -1),
        convs[0]["W"][_D:2 * _D],
    )

    for li, cp in enumerate(convs):
        w = cp["W"]
        ws, we = w[:_D], w[2 * _D:]
        wbig = jnp.kron(eye_m, we).astype(jnp.bfloat16)

        gv3 = _sc_gather(vb, fi_t, 128, True).reshape(m, n, _D2)

        s_p, q_p = _conv_stats(gv3, ef2, x, ws, wbig)
        summed, s2_p, q2_p = _conv_apply(
            gv3, ef2, x, ws, wbig, s_p, q_p,
            cp["g1"].reshape(1, -1), cp["b1"].reshape(1, -1), cnt,
        )
        wn_next = convs[li + 1]["W"][_D:2 * _D] if li + 1 < len(convs) else None
        res = _residual(
            x, summed, s2_p, q2_p,
            cp["g2"].reshape(1, -1), cp["b2"].reshape(1, -1), float(n),
            wn_next,
        )
        if wn_next is not None:
            x, vb = res
        else:
            x = res

    c, a = crystal_index.shape
    gc3 = _sc_gather(x, ci_t, 80, False).reshape(a, c, _D)
    return _head(
        gc3,
        p["W_fc"], p["b_fc"].reshape(1, -1),
        p["W_h1"], p["b_h1"].reshape(1, -1),
        p["W_h2"], p["b_h2"].reshape(1, -1),
    )
